# baseline (speedup 1.0000x reference)
import functools

import jax
import jax.numpy as jnp
from jax import lax
from jax.experimental import pallas as pl
from jax.experimental.pallas import tpu as pltpu
from jax.experimental.pallas import tpu_sc as plsc

B = 16384
MF_DIM = 16
MLP_DIM = 32

_NC = 2
_NS = 16
_NW = _NC * _NS
_BPW = B // _NW
_CH = 128
_NCH = _BPW // _CH

_BLK = 2048


def _sc_gather(uidx2, iidx2, mf_user, mf_item, mlp_user, mlp_item):
    mesh = plsc.VectorSubcoreMesh(core_axis_name="c", subcore_axis_name="s")

    @functools.partial(
        pl.kernel,
        mesh=mesh,
        compiler_params=pltpu.CompilerParams(use_tc_tiling_on_sc=False),
        out_type=[
            jax.ShapeDtypeStruct((B, MLP_DIM), jnp.float32),
            jax.ShapeDtypeStruct((B, MLP_DIM), jnp.float32),
            jax.ShapeDtypeStruct((B, MF_DIM), jnp.float32),
            jax.ShapeDtypeStruct((B, MF_DIM), jnp.float32),
        ],
        scratch_types=[
            pltpu.VMEM((_NCH, _CH), jnp.int32),
            pltpu.VMEM((_NCH, _CH), jnp.int32),
            pltpu.VMEM((_BPW, MLP_DIM), jnp.float32),
            pltpu.VMEM((_BPW, MLP_DIM), jnp.float32),
            pltpu.VMEM((_BPW, MF_DIM), jnp.float32),
            pltpu.VMEM((_BPW, MF_DIM), jnp.float32),
            pltpu.SemaphoreType.DMA,
        ],
    )
    def k(uidx_hbm, iidx_hbm, mfu_hbm, mfi_hbm, mlpu_hbm, mlpi_hbm,
          out_mlpu, out_mlpi, out_mfu, out_mfi,
          uidx_v, iidx_v, mlpu_v, mlpi_v, mfu_v, mfi_v, sem):
        wid = lax.axis_index("s") * _NC + lax.axis_index("c")
        row = wid * _NCH
        pltpu.sync_copy(uidx_hbm.at[pl.ds(row, _NCH)], uidx_v)
        pltpu.sync_copy(iidx_hbm.at[pl.ds(row, _NCH)], iidx_v)
        copies = []
        for j in range(_NCH):
            sl = pl.ds(j * _CH, _CH)
            copies.append(pltpu.async_copy(mlpu_hbm.at[uidx_v.at[j]], mlpu_v.at[sl], sem))
            copies.append(pltpu.async_copy(mlpi_hbm.at[iidx_v.at[j]], mlpi_v.at[sl], sem))
            copies.append(pltpu.async_copy(mfu_hbm.at[uidx_v.at[j]], mfu_v.at[sl], sem))
            copies.append(pltpu.async_copy(mfi_hbm.at[iidx_v.at[j]], mfi_v.at[sl], sem))
        for c in copies:
            c.wait()
        bs = pl.ds(wid * _BPW, _BPW)
        pltpu.sync_copy(mlpu_v, out_mlpu.at[bs])
        pltpu.sync_copy(mlpi_v, out_mlpi.at[bs])
        pltpu.sync_copy(mfu_v, out_mfu.at[bs])
        pltpu.sync_copy(mfi_v, out_mfi.at[bs])

    return k(uidx2, iidx2, mf_user, mf_item, mlp_user, mlp_item)


def _mlp_body(mlpu_ref, mlpi_ref, mfu_ref, mfi_ref,
              w1a_ref, w1b_ref, b1_ref, w2_ref, b2_ref,
              wph_ref, wpm_ref, bp_ref, out_ref):
    h = jnp.dot(mlpu_ref[...], w1a_ref[...], preferred_element_type=jnp.float32)
    h = h + jnp.dot(mlpi_ref[...], w1b_ref[...], preferred_element_type=jnp.float32)
    h = jnp.maximum(h + b1_ref[...], 0.0)
    h2 = jnp.dot(h, w2_ref[...], preferred_element_type=jnp.float32)
    h2 = jnp.maximum(h2 + b2_ref[...], 0.0)
    mf = mfu_ref[...] * mfi_ref[...]
    logit = jnp.sum(h2 * wph_ref[...] + mf * wpm_ref[...], axis=1, keepdims=True)
    out_ref[...] = jax.nn.sigmoid(logit + bp_ref[...])[:, 0]


def _mlp_call(mlpu, mlpi, mfu, mfi, w1a, w1b, b1, w2, b2, wph, wpm, bp):
    full = lambda shape: pl.BlockSpec(shape, lambda i: (0, 0))
    return pl.pallas_call(
        _mlp_body,
        grid=(B // _BLK,),
        in_specs=[
            pl.BlockSpec((_BLK, MLP_DIM), lambda i: (i, 0)),
            pl.BlockSpec((_BLK, MLP_DIM), lambda i: (i, 0)),
            pl.BlockSpec((_BLK, MF_DIM), lambda i: (i, 0)),
            pl.BlockSpec((_BLK, MF_DIM), lambda i: (i, 0)),
            full((MLP_DIM, MLP_DIM)),
            full((MLP_DIM, MLP_DIM)),
            full((1, MLP_DIM)),
            full((MLP_DIM, MF_DIM)),
            full((1, MF_DIM)),
            full((1, MF_DIM)),
            full((1, MF_DIM)),
            full((1, 1)),
        ],
        out_specs=pl.BlockSpec((_BLK,), lambda i: (i,)),
        out_shape=jax.ShapeDtypeStruct((B,), jnp.float32),
    )(mlpu, mlpi, mfu, mfi, w1a, w1b, b1, w2, b2, wph, wpm, bp)


def kernel(user_input, item_input, mf_user, mf_item, mlp_user, mlp_item,
           W1, b1, W2, b2, Wp, bp):
    uidx2 = user_input.astype(jnp.int32).reshape(B // _CH, _CH)
    iidx2 = item_input.astype(jnp.int32).reshape(B // _CH, _CH)
    zero = lax.optimization_barrier(jnp.float32(0))
    mf_user = mf_user + zero
    mf_item = mf_item + zero
    mlp_user = mlp_user + zero
    mlp_item = mlp_item + zero
    mlpu, mlpi, mfu, mfi = _sc_gather(uidx2, iidx2, mf_user, mf_item,
                                      mlp_user, mlp_item)
    w1a = W1[:, :MLP_DIM].T
    w1b = W1[:, MLP_DIM:].T
    return _mlp_call(mlpu, mlpi, mfu, mfi,
                     w1a, w1b, b1.reshape(1, MLP_DIM),
                     W2.T, b2.reshape(1, MF_DIM),
                     Wp[:, :MF_DIM], Wp[:, MF_DIM:], bp.reshape(1, 1))


# speedup vs baseline: 5.5193x; 5.5193x over previous
import functools

import jax
import jax.numpy as jnp
from jax import lax
from jax.experimental import pallas as pl
from jax.experimental.pallas import tpu as pltpu
from jax.experimental.pallas import tpu_sc as plsc

B = 16384
V = 1000000
MF_DIM = 16
MLP_DIM = 32

_NC = 2
_NS = 16
_NW = _NC * _NS
_BPW = B // _NW
_G = 16
_SG = 4

_BLK = 2048


def _sc_gather(uidx, iidx, mfu3, mfi3, mlpu3, mlpi3):
    mesh = plsc.VectorSubcoreMesh(core_axis_name="c", subcore_axis_name="s")

    @functools.partial(
        pl.kernel,
        mesh=mesh,
        out_type=[
            jax.ShapeDtypeStruct((MLP_DIM, B), jnp.float32),
            jax.ShapeDtypeStruct((MLP_DIM, B), jnp.float32),
            jax.ShapeDtypeStruct((MF_DIM, B), jnp.float32),
            jax.ShapeDtypeStruct((MF_DIM, B), jnp.float32),
        ],
        scratch_types=[
            pltpu.VMEM((_BPW,), jnp.int32),
            pltpu.VMEM((_BPW,), jnp.int32),
            pltpu.VMEM((MLP_DIM, _BPW), jnp.float32),
            pltpu.VMEM((MLP_DIM, _BPW), jnp.float32),
            pltpu.VMEM((MF_DIM, _BPW), jnp.float32),
            pltpu.VMEM((MF_DIM, _BPW), jnp.float32),
            pltpu.VMEM_SHARED((_NS, _SG, MLP_DIM // 8, 8, 128), jnp.float32),
            pltpu.VMEM_SHARED((_NS, _SG, MLP_DIM // 8, 8, 128), jnp.float32),
            pltpu.VMEM_SHARED((_NS, _SG, MF_DIM // 8, 8, 128), jnp.float32),
            pltpu.VMEM_SHARED((_NS, _SG, MF_DIM // 8, 8, 128), jnp.float32),
            pltpu.SemaphoreType.DMA,
        ],
    )
    def k(uidx_hbm, iidx_hbm, mfu_hbm, mfi_hbm, mlpu_hbm, mlpi_hbm,
          out_mlpu, out_mlpi, out_mfu, out_mfi,
          uidx_v, iidx_v, mlpu_v, mlpi_v, mfu_v, mfi_v,
          stg_mlpu, stg_mlpi, stg_mfu, stg_mfi, sem):
        sid = lax.axis_index("s")
        wid = sid * _NC + lax.axis_index("c")
        base = wid * _BPW
        pltpu.sync_copy(uidx_hbm.at[pl.ds(base, _BPW)], uidx_v)
        pltpu.sync_copy(iidx_hbm.at[pl.ds(base, _BPW)], iidx_v)

        w16 = pl.ds(0, 16)

        def group(g, carry):
            ug = uidx_v[pl.ds(g * _G, _G)]
            vg = iidx_v[pl.ds(g * _G, _G)]
            uq16 = ug >> 4
            vq16 = vg >> 4
            ucol = ug & jnp.int32(15)
            vcol = vg & jnp.int32(15)
            for s in range(_G // _SG):
                copies = []
                for l in range(s * _SG, (s + 1) * _SG):
                    sl = l - s * _SG
                    uq = lax.index_in_dim(uq16, l, 0, keepdims=False)
                    vq = lax.index_in_dim(vq16, l, 0, keepdims=False)
                    uw = pl.ds(uq * 16, 16)
                    vw = pl.ds(vq * 16, 16)
                    for a in range(MLP_DIM // 8):
                        copies.append(pltpu.async_copy(
                            mlpu_hbm.at[a, :, uw],
                            stg_mlpu.at[sid, sl, a, :, w16], sem))
                        copies.append(pltpu.async_copy(
                            mlpi_hbm.at[a, :, vw],
                            stg_mlpi.at[sid, sl, a, :, w16], sem))
                    for a in range(MF_DIM // 8):
                        copies.append(pltpu.async_copy(
                            mfu_hbm.at[a, :, uw],
                            stg_mfu.at[sid, sl, a, :, w16], sem))
                        copies.append(pltpu.async_copy(
                            mfi_hbm.at[a, :, vw],
                            stg_mfi.at[sid, sl, a, :, w16], sem))
                for c in copies:
                    c.wait()
                copies2 = []
                for l in range(s * _SG, (s + 1) * _SG):
                    sl = l - s * _SG
                    li = g * _G + l
                    uc = lax.index_in_dim(ucol, l, 0, keepdims=False)
                    vc = lax.index_in_dim(vcol, l, 0, keepdims=False)
                    for a in range(MLP_DIM // 8):
                        fs = pl.ds(a * 8, 8)
                        copies2.append(pltpu.async_copy(
                            stg_mlpu.at[sid, sl, a, :, uc],
                            mlpu_v.at[fs, li], sem))
                        copies2.append(pltpu.async_copy(
                            stg_mlpi.at[sid, sl, a, :, vc],
                            mlpi_v.at[fs, li], sem))
                    for a in range(MF_DIM // 8):
                        fs = pl.ds(a * 8, 8)
                        copies2.append(pltpu.async_copy(
                            stg_mfu.at[sid, sl, a, :, uc],
                            mfu_v.at[fs, li], sem))
                        copies2.append(pltpu.async_copy(
                            stg_mfi.at[sid, sl, a, :, vc],
                            mfi_v.at[fs, li], sem))
                for c in copies2:
                    c.wait()
            return carry

        lax.fori_loop(0, _BPW // _G, group, 0)
        bs = pl.ds(base, _BPW)
        pltpu.sync_copy(mlpu_v, out_mlpu.at[:, bs])
        pltpu.sync_copy(mlpi_v, out_mlpi.at[:, bs])
        pltpu.sync_copy(mfu_v, out_mfu.at[:, bs])
        pltpu.sync_copy(mfi_v, out_mfi.at[:, bs])

    return k(uidx, iidx, mfu3, mfi3, mlpu3, mlpi3)


def _mlp_body(mlpu_ref, mlpi_ref, mfu_ref, mfi_ref,
              w1a_ref, w1b_ref, b1_ref, w2_ref, b2_ref,
              wph_ref, wpm_ref, bp_ref, out_ref):
    h = jnp.dot(w1a_ref[...], mlpu_ref[...], preferred_element_type=jnp.float32)
    h = h + jnp.dot(w1b_ref[...], mlpi_ref[...], preferred_element_type=jnp.float32)
    h = jnp.maximum(h + b1_ref[...], 0.0)
    h2 = jnp.dot(w2_ref[...], h, preferred_element_type=jnp.float32)
    h2 = jnp.maximum(h2 + b2_ref[...], 0.0)
    mf = mfu_ref[...] * mfi_ref[...]
    logit = jnp.dot(wph_ref[...], h2, preferred_element_type=jnp.float32)
    logit = logit + jnp.dot(wpm_ref[...], mf, preferred_element_type=jnp.float32)
    out_ref[...] = jax.nn.sigmoid(logit + bp_ref[...])[0, :]


def _mlp_call(mlpu_t, mlpi_t, mfu_t, mfi_t, w1a, w1b, b1, w2, b2, wph, wpm, bp):
    full = lambda shape: pl.BlockSpec(shape, lambda i: (0, 0))
    return pl.pallas_call(
        _mlp_body,
        grid=(B // _BLK,),
        in_specs=[
            pl.BlockSpec((MLP_DIM, _BLK), lambda i: (0, i)),
            pl.BlockSpec((MLP_DIM, _BLK), lambda i: (0, i)),
            pl.BlockSpec((MF_DIM, _BLK), lambda i: (0, i)),
            pl.BlockSpec((MF_DIM, _BLK), lambda i: (0, i)),
            full((MLP_DIM, MLP_DIM)),
            full((MLP_DIM, MLP_DIM)),
            full((MLP_DIM, 1)),
            full((MF_DIM, MLP_DIM)),
            full((MF_DIM, 1)),
            full((1, MF_DIM)),
            full((1, MF_DIM)),
            full((1, 1)),
        ],
        out_specs=pl.BlockSpec((_BLK,), lambda i: (i,)),
        out_shape=jax.ShapeDtypeStruct((B,), jnp.float32),
    )(mlpu_t, mlpi_t, mfu_t, mfi_t, w1a, w1b, b1, w2, b2, wph, wpm, bp)


def kernel(user_input, item_input, mf_user, mf_item, mlp_user, mlp_item,
           W1, b1, W2, b2, Wp, bp):
    uidx = user_input.astype(jnp.int32)
    iidx = item_input.astype(jnp.int32)
    mfu3 = mf_user.T.reshape(MF_DIM // 8, 8, V)
    mfi3 = mf_item.T.reshape(MF_DIM // 8, 8, V)
    mlpu3 = mlp_user.T.reshape(MLP_DIM // 8, 8, V)
    mlpi3 = mlp_item.T.reshape(MLP_DIM // 8, 8, V)
    mlpu_t, mlpi_t, mfu_t, mfi_t = _sc_gather(uidx, iidx, mfu3, mfi3,
                                              mlpu3, mlpi3)
    w1a = W1[:, :MLP_DIM]
    w1b = W1[:, MLP_DIM:]
    return _mlp_call(mlpu_t, mlpi_t, mfu_t, mfi_t,
                     w1a, w1b, b1.reshape(MLP_DIM, 1),
                     W2, b2.reshape(MF_DIM, 1),
                     Wp[:, :MF_DIM], Wp[:, MF_DIM:], bp.reshape(1, 1))


# speedup vs baseline: 6.0094x; 1.0888x over previous
import functools

import jax
import jax.numpy as jnp
from jax import lax
from jax.experimental import pallas as pl
from jax.experimental.pallas import tpu as pltpu
from jax.experimental.pallas import tpu_sc as plsc

B = 16384
V = 1000000
MF_DIM = 16
MLP_DIM = 32

_NC = 2
_NS = 16
_NW = _NC * _NS
_BPW = B // _NW
_G = 16
_SG = 8

_BLK = 2048


def _sc_gather(uidx, iidx, mfu3, mfi3, mlpu3, mlpi3):
    mesh = plsc.VectorSubcoreMesh(core_axis_name="c", subcore_axis_name="s")

    @functools.partial(
        pl.kernel,
        mesh=mesh,
        out_type=[
            jax.ShapeDtypeStruct((MLP_DIM, B), jnp.float32),
            jax.ShapeDtypeStruct((MLP_DIM, B), jnp.float32),
            jax.ShapeDtypeStruct((MF_DIM, B), jnp.float32),
            jax.ShapeDtypeStruct((MF_DIM, B), jnp.float32),
        ],
        scratch_types=[
            pltpu.VMEM((_BPW,), jnp.int32),
            pltpu.VMEM((_BPW,), jnp.int32),
            pltpu.VMEM((MLP_DIM, _BPW), jnp.float32),
            pltpu.VMEM((MLP_DIM, _BPW), jnp.float32),
            pltpu.VMEM((MF_DIM, _BPW), jnp.float32),
            pltpu.VMEM((MF_DIM, _BPW), jnp.float32),
            pltpu.VMEM_SHARED((_NS, _SG, MLP_DIM // 8, 8, 128), jnp.float32),
            pltpu.VMEM_SHARED((_NS, _SG, MLP_DIM // 8, 8, 128), jnp.float32),
            pltpu.SemaphoreType.DMA,
        ],
    )
    def k(uidx_hbm, iidx_hbm, mfu_hbm, mfi_hbm, mlpu_hbm, mlpi_hbm,
          out_mlpu, out_mlpi, out_mfu, out_mfi,
          uidx_v, iidx_v, mlpu_v, mlpi_v, mfu_v, mfi_v,
          stg_mlpu, stg_mlpi, sem):
        sid = lax.axis_index("s")
        wid = sid * _NC + lax.axis_index("c")
        base = wid * _BPW
        pltpu.sync_copy(uidx_hbm.at[pl.ds(base, _BPW)], uidx_v)
        pltpu.sync_copy(iidx_hbm.at[pl.ds(base, _BPW)], iidx_v)

        w16 = pl.ds(0, 16)
        wmf = pl.ds(16, 16)

        def group(g, carry):
            ug = uidx_v[pl.ds(g * _G, _G)]
            vg = iidx_v[pl.ds(g * _G, _G)]
            uq16 = ug >> 4
            vq16 = vg >> 4
            ucol = ug & jnp.int32(15)
            vcol = vg & jnp.int32(15)
            for s in range(_G // _SG):
                copies = []
                for l in range(s * _SG, (s + 1) * _SG):
                    sl = l - s * _SG
                    uq = lax.index_in_dim(uq16, l, 0, keepdims=False)
                    vq = lax.index_in_dim(vq16, l, 0, keepdims=False)
                    uw = pl.ds(uq * 16, 16)
                    vw = pl.ds(vq * 16, 16)
                    for a in range(MLP_DIM // 8):
                        copies.append(pltpu.async_copy(
                            mlpu_hbm.at[a, :, uw],
                            stg_mlpu.at[sid, sl, a, :, w16], sem))
                        copies.append(pltpu.async_copy(
                            mlpi_hbm.at[a, :, vw],
                            stg_mlpi.at[sid, sl, a, :, w16], sem))
                    for a in range(MF_DIM // 8):
                        copies.append(pltpu.async_copy(
                            mfu_hbm.at[a, :, uw],
                            stg_mlpu.at[sid, sl, a, :, wmf], sem))
                        copies.append(pltpu.async_copy(
                            mfi_hbm.at[a, :, vw],
                            stg_mlpi.at[sid, sl, a, :, wmf], sem))
                for c in copies:
                    c.wait()
                copies2 = []
                for l in range(s * _SG, (s + 1) * _SG):
                    sl = l - s * _SG
                    li = g * _G + l
                    uc = lax.index_in_dim(ucol, l, 0, keepdims=False)
                    vc = lax.index_in_dim(vcol, l, 0, keepdims=False)
                    for a in range(MLP_DIM // 8):
                        fs = pl.ds(a * 8, 8)
                        copies2.append(pltpu.async_copy(
                            stg_mlpu.at[sid, sl, a, :, uc],
                            mlpu_v.at[fs, li], sem))
                        copies2.append(pltpu.async_copy(
                            stg_mlpi.at[sid, sl, a, :, vc],
                            mlpi_v.at[fs, li], sem))
                    ucm = uc + jnp.int32(16)
                    vcm = vc + jnp.int32(16)
                    for a in range(MF_DIM // 8):
                        fs = pl.ds(a * 8, 8)
                        copies2.append(pltpu.async_copy(
                            stg_mlpu.at[sid, sl, a, :, ucm],
                            mfu_v.at[fs, li], sem))
                        copies2.append(pltpu.async_copy(
                            stg_mlpi.at[sid, sl, a, :, vcm],
                            mfi_v.at[fs, li], sem))
                for c in copies2:
                    c.wait()
            return carry

        lax.fori_loop(0, _BPW // _G, group, 0)
        bs = pl.ds(base, _BPW)
        pltpu.sync_copy(mlpu_v, out_mlpu.at[:, bs])
        pltpu.sync_copy(mlpi_v, out_mlpi.at[:, bs])
        pltpu.sync_copy(mfu_v, out_mfu.at[:, bs])
        pltpu.sync_copy(mfi_v, out_mfi.at[:, bs])

    return k(uidx, iidx, mfu3, mfi3, mlpu3, mlpi3)


def _mlp_body(mlpu_ref, mlpi_ref, mfu_ref, mfi_ref,
              w1a_ref, w1b_ref, b1_ref, w2_ref, b2_ref,
              wph_ref, wpm_ref, bp_ref, out_ref):
    h = jnp.dot(w1a_ref[...], mlpu_ref[...], preferred_element_type=jnp.float32)
    h = h + jnp.dot(w1b_ref[...], mlpi_ref[...], preferred_element_type=jnp.float32)
    h = jnp.maximum(h + b1_ref[...], 0.0)
    h2 = jnp.dot(w2_ref[...], h, preferred_element_type=jnp.float32)
    h2 = jnp.maximum(h2 + b2_ref[...], 0.0)
    mf = mfu_ref[...] * mfi_ref[...]
    logit = jnp.dot(wph_ref[...], h2, preferred_element_type=jnp.float32)
    logit = logit + jnp.dot(wpm_ref[...], mf, preferred_element_type=jnp.float32)
    out_ref[...] = jax.nn.sigmoid(logit + bp_ref[...])[0, :]


def _mlp_call(mlpu_t, mlpi_t, mfu_t, mfi_t, w1a, w1b, b1, w2, b2, wph, wpm, bp):
    full = lambda shape: pl.BlockSpec(shape, lambda i: (0, 0))
    return pl.pallas_call(
        _mlp_body,
        grid=(B // _BLK,),
        in_specs=[
            pl.BlockSpec((MLP_DIM, _BLK), lambda i: (0, i)),
            pl.BlockSpec((MLP_DIM, _BLK), lambda i: (0, i)),
            pl.BlockSpec((MF_DIM, _BLK), lambda i: (0, i)),
            pl.BlockSpec((MF_DIM, _BLK), lambda i: (0, i)),
            full((MLP_DIM, MLP_DIM)),
            full((MLP_DIM, MLP_DIM)),
            full((MLP_DIM, 1)),
            full((MF_DIM, MLP_DIM)),
            full((MF_DIM, 1)),
            full((1, MF_DIM)),
            full((1, MF_DIM)),
            full((1, 1)),
        ],
        out_specs=pl.BlockSpec((_BLK,), lambda i: (i,)),
        out_shape=jax.ShapeDtypeStruct((B,), jnp.float32),
    )(mlpu_t, mlpi_t, mfu_t, mfi_t, w1a, w1b, b1, w2, b2, wph, wpm, bp)


def kernel(user_input, item_input, mf_user, mf_item, mlp_user, mlp_item,
           W1, b1, W2, b2, Wp, bp):
    uidx = user_input.astype(jnp.int32)
    iidx = item_input.astype(jnp.int32)
    mfu3 = mf_user.T.reshape(MF_DIM // 8, 8, V)
    mfi3 = mf_item.T.reshape(MF_DIM // 8, 8, V)
    mlpu3 = mlp_user.T.reshape(MLP_DIM // 8, 8, V)
    mlpi3 = mlp_item.T.reshape(MLP_DIM // 8, 8, V)
    mlpu_t, mlpi_t, mfu_t, mfi_t = _sc_gather(uidx, iidx, mfu3, mfi3,
                                              mlpu3, mlpi3)
    w1a = W1[:, :MLP_DIM]
    w1b = W1[:, MLP_DIM:]
    return _mlp_call(mlpu_t, mlpi_t, mfu_t, mfi_t,
                     w1a, w1b, b1.reshape(MLP_DIM, 1),
                     W2, b2.reshape(MF_DIM, 1),
                     Wp[:, :MF_DIM], Wp[:, MF_DIM:], bp.reshape(1, 1))
